# baseline (speedup 1.0000x reference)
import functools

import numpy as np

import jax
import jax.numpy as jnp
from jax.experimental import pallas as pl
from jax.experimental.pallas import tpu as pltpu

try:
    from jax.experimental.shard_map import shard_map as _shard_map
except ImportError:
    from jax import shard_map as _shard_map
from jax.sharding import Mesh, PartitionSpec as _P

_NEG = -1e30


def _ctx_kernel(axt_ref, ayt_ref, nx_ref, ny_ref, nf_ref, q_ref, out_ref,
                pen_ref, *, dist_sq, ns):
    a_total = axt_ref.shape[1]
    n_total = nx_ref.shape[0]
    h = nf_ref.shape[1]
    n_groups = a_total // 8
    n_slabs = n_total // ns

    def p1_body(s, carry):
        s0 = s * ns
        dx = nx_ref[pl.ds(s0, ns), :] - axt_ref[...]
        dy = ny_ref[pl.ds(s0, ns), :] - ayt_ref[...]
        pen = jnp.where(dx * dx + dy * dy <= dist_sq, 0.0, _NEG)
        pen_ref[pl.ds(s0, ns), :] = pen.astype(jnp.bfloat16)
        return carry

    jax.lax.fori_loop(0, n_slabs, p1_body, 0)

    def group_body(g, carry):
        qg = q_ref[pl.ds(a_total - 8 - 8 * g, a_total), :]

        accs = [jnp.full((8, h), _NEG, jnp.float32) for _ in range(8)]
        for s in range(n_slabs):
            s0 = s * ns
            exp = jax.lax.dot_general(
                pen_ref[pl.ds(s0, ns), :], qg, (((1,), (0,)), ((), ())),
                preferred_element_type=jnp.float32)
            nf_s = nf_ref[pl.ds(s0, ns), :]
            for j in range(8):
                seg = jax.lax.slice(exp, (0, j * h), (ns, (j + 1) * h))
                masked = nf_s + seg
                fold = jnp.max(masked.reshape(ns // 8, 8, h), axis=0)
                accs[j] = jnp.maximum(accs[j], fold)

        rows = [jnp.max(a, axis=0, keepdims=True) for a in accs]
        red = jnp.concatenate(rows, axis=0)
        out_ref[pl.ds(g * 8, 8), :] = jnp.where(red > 0.5 * _NEG, red, 0.0)
        return carry

    jax.lax.fori_loop(0, n_groups, group_body, 0)


def _forward(actor_ctrs, node_ctrs, node_feats):
    B, A, _ = actor_ctrs.shape
    _, N, H = node_feats.shape
    dist_th = 6.0

    f32 = jnp.float32
    axt = actor_ctrs[..., 0].astype(f32).reshape(B, 1, A)
    ayt = actor_ctrs[..., 1].astype(f32).reshape(B, 1, A)
    nx = node_ctrs[..., 0:1].astype(f32)
    ny = node_ctrs[..., 1:2].astype(f32)
    nf = node_feats.astype(f32)

    r = jnp.arange(2 * A, dtype=jnp.int32)[:, None]
    c = jnp.arange(8 * H, dtype=jnp.int32)[None, :]
    q = ((r - (A - 8)) == (c // H)).astype(jnp.bfloat16)

    NS = 512 if N % 512 == 0 else N

    kern = functools.partial(_ctx_kernel, dist_sq=float(dist_th) ** 2, ns=NS)
    ctx = pl.pallas_call(
        kern,
        out_shape=jax.ShapeDtypeStruct((B, A, H), jnp.float32),
        grid=(B,),
        in_specs=[
            pl.BlockSpec((None, 1, A), lambda b: (b, 0, 0)),
            pl.BlockSpec((None, 1, A), lambda b: (b, 0, 0)),
            pl.BlockSpec((None, N, 1), lambda b: (b, 0, 0)),
            pl.BlockSpec((None, N, 1), lambda b: (b, 0, 0)),
            pl.BlockSpec((None, N, H), lambda b: (b, 0, 0)),
            pl.BlockSpec((2 * A, 8 * H), lambda b: (0, 0)),
        ],
        out_specs=pl.BlockSpec((None, A, H), lambda b: (b, 0, 0)),
        scratch_shapes=[pltpu.VMEM((N, A), jnp.bfloat16)],
        compiler_params=pltpu.CompilerParams(
            dimension_semantics=("parallel",),
            vmem_limit_bytes=48 << 20),
    )(axt, ayt, nx, ny, nf, q)

    return ctx.reshape(B * A, H)


def kernel(actor_ctrs, node_ctrs, node_feats):
    devs = jax.devices()
    B = actor_ctrs.shape[0]
    nd = 1
    for d in range(min(len(devs), B), 0, -1):
        if B % d == 0:
            nd = d
            break
    if nd > 1:
        mesh = Mesh(np.asarray(devs[:nd]), ("d",))
        fwd = _shard_map(_forward, mesh=mesh,
                         in_specs=(_P("d"), _P("d"), _P("d")),
                         out_specs=_P("d"), check_rep=False)
        return fwd(actor_ctrs, node_ctrs, node_feats)
    return _forward(actor_ctrs, node_ctrs, node_feats)


# speedup vs baseline: 1.0339x; 1.0339x over previous
import functools

import numpy as np

import jax
import jax.numpy as jnp
from jax.experimental import pallas as pl
from jax.experimental.pallas import tpu as pltpu

try:
    from jax.experimental.shard_map import shard_map as _shard_map
except ImportError:
    from jax import shard_map as _shard_map
from jax.sharding import Mesh, PartitionSpec as _P

_NEG = -1e30


def _ctx_kernel(axt_ref, ayt_ref, nx_ref, ny_ref, nf_ref, q_ref, out_ref,
                pen_ref, *, dist_sq, ns):
    a_total = axt_ref.shape[1]
    n_total = nx_ref.shape[0]
    h = nf_ref.shape[1]
    n_groups = a_total // 8
    n_slabs = n_total // ns

    def p1_body(s, carry):
        s0 = s * ns
        dx = nx_ref[pl.ds(s0, ns), :] - axt_ref[...]
        dy = ny_ref[pl.ds(s0, ns), :] - ayt_ref[...]
        pen = jnp.where(dx * dx + dy * dy <= dist_sq, 0.0, _NEG)
        pen_ref[pl.ds(s0, ns), :] = pen.astype(jnp.bfloat16)
        return carry

    jax.lax.fori_loop(0, n_slabs, p1_body, 0)

    def group_body(gp, carry):
        for half in range(2):
            g = gp * 2 + half
            qg = q_ref[pl.ds(a_total - 8 - 8 * g, a_total), :]

            accs = [jnp.full((8, h), _NEG, jnp.float32) for _ in range(8)]
            for s in range(n_slabs):
                s0 = s * ns
                exp = jax.lax.dot_general(
                    pen_ref[pl.ds(s0, ns), :], qg, (((1,), (0,)), ((), ())),
                    preferred_element_type=jnp.float32)
                nf_s = nf_ref[pl.ds(s0, ns), :]
                for j in range(8):
                    seg = jax.lax.slice(exp, (0, j * h),
                                        (ns, (j + 1) * h))
                    masked = nf_s + seg
                    fold = jnp.max(masked.reshape(ns // 8, 8, h), axis=0)
                    accs[j] = jnp.maximum(accs[j], fold)

            rows = [jnp.max(a, axis=0, keepdims=True) for a in accs]
            red = jnp.concatenate(rows, axis=0)
            out_ref[pl.ds(g * 8, 8), :] = jnp.where(red > 0.5 * _NEG,
                                                    red, 0.0)
        return carry

    jax.lax.fori_loop(0, n_groups // 2, group_body, 0)


def _forward(actor_ctrs, node_ctrs, node_feats):
    B, A, _ = actor_ctrs.shape
    _, N, H = node_feats.shape
    dist_th = 6.0

    f32 = jnp.float32
    axt = actor_ctrs[..., 0].astype(f32).reshape(B, 1, A)
    ayt = actor_ctrs[..., 1].astype(f32).reshape(B, 1, A)
    nx = node_ctrs[..., 0:1].astype(f32)
    ny = node_ctrs[..., 1:2].astype(f32)
    nf = node_feats.astype(f32)

    r = jnp.arange(2 * A, dtype=jnp.int32)[:, None]
    c = jnp.arange(8 * H, dtype=jnp.int32)[None, :]
    q = ((r - (A - 8)) == (c // H)).astype(jnp.bfloat16)

    NS = 512 if N % 512 == 0 else N

    kern = functools.partial(_ctx_kernel, dist_sq=float(dist_th) ** 2, ns=NS)
    ctx = pl.pallas_call(
        kern,
        out_shape=jax.ShapeDtypeStruct((B, A, H), jnp.float32),
        grid=(B,),
        in_specs=[
            pl.BlockSpec((None, 1, A), lambda b: (b, 0, 0)),
            pl.BlockSpec((None, 1, A), lambda b: (b, 0, 0)),
            pl.BlockSpec((None, N, 1), lambda b: (b, 0, 0)),
            pl.BlockSpec((None, N, 1), lambda b: (b, 0, 0)),
            pl.BlockSpec((None, N, H), lambda b: (b, 0, 0)),
            pl.BlockSpec((2 * A, 8 * H), lambda b: (0, 0)),
        ],
        out_specs=pl.BlockSpec((None, A, H), lambda b: (b, 0, 0)),
        scratch_shapes=[pltpu.VMEM((N, A), jnp.bfloat16)],
        compiler_params=pltpu.CompilerParams(
            dimension_semantics=("parallel",),
            vmem_limit_bytes=48 << 20),
    )(axt, ayt, nx, ny, nf, q)

    return ctx.reshape(B * A, H)


def kernel(actor_ctrs, node_ctrs, node_feats):
    devs = jax.devices()
    B = actor_ctrs.shape[0]
    nd = 1
    for d in range(min(len(devs), B), 0, -1):
        if B % d == 0:
            nd = d
            break
    if nd > 1:
        mesh = Mesh(np.asarray(devs[:nd]), ("d",))
        fwd = _shard_map(_forward, mesh=mesh,
                         in_specs=(_P("d"), _P("d"), _P("d")),
                         out_specs=_P("d"), check_rep=False)
        return fwd(actor_ctrs, node_ctrs, node_feats)
    return _forward(actor_ctrs, node_ctrs, node_feats)


# speedup vs baseline: 1.0725x; 1.0374x over previous
import functools

import numpy as np

import jax
import jax.numpy as jnp
from jax.experimental import pallas as pl
from jax.experimental.pallas import tpu as pltpu

try:
    from jax.experimental.shard_map import shard_map as _shard_map
except ImportError:
    from jax import shard_map as _shard_map
from jax.sharding import Mesh, PartitionSpec as _P

_NEG = -1e30


def _ctx_kernel(axt_ref, ayt_ref, nx_ref, ny_ref, nf_ref, q_ref, out_ref,
                pen_ref, *, dist_sq, ns):
    a_total = axt_ref.shape[1]
    n_total = nx_ref.shape[0]
    h = nf_ref.shape[1]
    n_groups = a_total // 8
    n_slabs = n_total // ns

    def p1_body(s, carry):
        s0 = s * ns
        dx = nx_ref[pl.ds(s0, ns), :] - axt_ref[...]
        dy = ny_ref[pl.ds(s0, ns), :] - ayt_ref[...]
        pen = jnp.where(dx * dx + dy * dy <= dist_sq, 0.0, _NEG)
        pen_ref[pl.ds(s0, ns), :] = pen.astype(jnp.bfloat16)
        return carry

    jax.lax.fori_loop(0, n_slabs, p1_body, 0)

    def group_body(gp, carry):
        for half in range(4):
            g = gp * 4 + half
            qg = q_ref[pl.ds(a_total - 8 - 8 * g, a_total), :]

            accs = [jnp.full((8, h), _NEG, jnp.float32) for _ in range(8)]
            for s in range(n_slabs):
                s0 = s * ns
                exp = jax.lax.dot_general(
                    pen_ref[pl.ds(s0, ns), :], qg, (((1,), (0,)), ((), ())),
                    preferred_element_type=jnp.float32)
                nf_s = nf_ref[pl.ds(s0, ns), :]
                for j in range(8):
                    seg = jax.lax.slice(exp, (0, j * h),
                                        (ns, (j + 1) * h))
                    masked = nf_s + seg
                    fold = jnp.max(masked.reshape(ns // 8, 8, h), axis=0)
                    accs[j] = jnp.maximum(accs[j], fold)

            rows = [jnp.max(a, axis=0, keepdims=True) for a in accs]
            red = jnp.concatenate(rows, axis=0)
            out_ref[pl.ds(g * 8, 8), :] = jnp.where(red > 0.5 * _NEG,
                                                    red, 0.0)
        return carry

    jax.lax.fori_loop(0, n_groups // 4, group_body, 0)


def _forward(actor_ctrs, node_ctrs, node_feats):
    B, A, _ = actor_ctrs.shape
    _, N, H = node_feats.shape
    dist_th = 6.0

    f32 = jnp.float32
    axt = actor_ctrs[..., 0].astype(f32).reshape(B, 1, A)
    ayt = actor_ctrs[..., 1].astype(f32).reshape(B, 1, A)
    nx = node_ctrs[..., 0:1].astype(f32)
    ny = node_ctrs[..., 1:2].astype(f32)
    nf = node_feats.astype(f32)

    r = jnp.arange(2 * A, dtype=jnp.int32)[:, None]
    c = jnp.arange(8 * H, dtype=jnp.int32)[None, :]
    q = ((r - (A - 8)) == (c // H)).astype(jnp.bfloat16)

    NS = 512 if N % 512 == 0 else N

    kern = functools.partial(_ctx_kernel, dist_sq=float(dist_th) ** 2, ns=NS)
    ctx = pl.pallas_call(
        kern,
        out_shape=jax.ShapeDtypeStruct((B, A, H), jnp.float32),
        grid=(B,),
        in_specs=[
            pl.BlockSpec((None, 1, A), lambda b: (b, 0, 0)),
            pl.BlockSpec((None, 1, A), lambda b: (b, 0, 0)),
            pl.BlockSpec((None, N, 1), lambda b: (b, 0, 0)),
            pl.BlockSpec((None, N, 1), lambda b: (b, 0, 0)),
            pl.BlockSpec((None, N, H), lambda b: (b, 0, 0)),
            pl.BlockSpec((2 * A, 8 * H), lambda b: (0, 0)),
        ],
        out_specs=pl.BlockSpec((None, A, H), lambda b: (b, 0, 0)),
        scratch_shapes=[pltpu.VMEM((N, A), jnp.bfloat16)],
        compiler_params=pltpu.CompilerParams(
            dimension_semantics=("parallel",),
            vmem_limit_bytes=48 << 20),
    )(axt, ayt, nx, ny, nf, q)

    return ctx.reshape(B * A, H)


def kernel(actor_ctrs, node_ctrs, node_feats):
    devs = jax.devices()
    B = actor_ctrs.shape[0]
    nd = 1
    for d in range(min(len(devs), B), 0, -1):
        if B % d == 0:
            nd = d
            break
    if nd > 1:
        mesh = Mesh(np.asarray(devs[:nd]), ("d",))
        fwd = _shard_map(_forward, mesh=mesh,
                         in_specs=(_P("d"), _P("d"), _P("d")),
                         out_specs=_P("d"), check_rep=False)
        return fwd(actor_ctrs, node_ctrs, node_feats)
    return _forward(actor_ctrs, node_ctrs, node_feats)
